# Initial kernel scaffold; baseline (speedup 1.0000x reference)
import jax
import jax.numpy as jnp
from jax.experimental import pallas as pl


def kernel(x, edge_index, pair_index, W1, b1, W2, b2, Wfc, bfc):
    raise NotImplementedError("write your pallas kernel here")


# speedup vs baseline: 28.9711x; 28.9711x over previous
import functools

import jax
import jax.numpy as jnp
from jax import lax
from jax.experimental import pallas as pl
from jax.experimental.pallas import tpu as pltpu
from jax.experimental.pallas import tpu_sc as plsc

NN = 10000
EE = 320000
PP = 100000

NC = 2
NS = 16
NW = NC * NS

E_PER_W = EE // NW
CHUNK = 100
N_CHUNKS = E_PER_W // CHUNK
R_MAIN = 632
R_LAST = NN - (NS - 1) * R_MAIN

@functools.cache
def _get_mesh():
    return plsc.VectorSubcoreMesh(
        core_axis_name="c", subcore_axis_name="s", num_cores=NC, num_subcores=NS
    )


NB = 5
GROUPS = N_CHUNKS // NB


@functools.cache
def _get_deg_kernel():
    return functools.partial(
        pl.kernel,
        out_type=jax.ShapeDtypeStruct((NC, NN, 16), jnp.float32),
        mesh=_get_mesh(),
        scratch_types=[
            pltpu.VMEM_SHARED((NN, 16), jnp.float32),
            pltpu.VMEM((R_MAIN, 16), jnp.float32),
            pltpu.VMEM((CHUNK, 16), jnp.float32),
            pltpu.VMEM((N_CHUNKS, CHUNK), jnp.int32),
            pltpu.SemaphoreType.DMA,
        ],
        compiler_params=pltpu.CompilerParams(use_tc_tiling_on_sc=False),
    )(_deg_body)


def _deg_body(dst_hbm, out_hbm, deg_sp, zbuf, ones, didx, sem):
    cid = lax.axis_index("c")
    sid = lax.axis_index("s")
    wid = cid * NS + sid

    z16 = jnp.zeros((16,), jnp.float32)
    o16 = jnp.ones((16,), jnp.float32)

    def fill_z(i, _):
        zbuf[i, :] = z16
        return 0

    lax.fori_loop(0, R_MAIN, fill_z, 0)

    def fill_o(i, _):
        ones[i, :] = o16
        return 0

    lax.fori_loop(0, CHUNK, fill_o, 0)

    pltpu.sync_copy(dst_hbm.at[wid], didx)

    @pl.when(sid < NS - 1)
    def _zmain():
        pltpu.sync_copy(zbuf, deg_sp.at[pl.ds(sid * R_MAIN, R_MAIN)])

    @pl.when(sid == NS - 1)
    def _zlast():
        pltpu.sync_copy(
            zbuf.at[pl.ds(0, R_LAST)],
            deg_sp.at[pl.ds((NS - 1) * R_MAIN, R_LAST)],
        )

    plsc.subcore_barrier()

    def step(g, _):
        for b in range(NB):
            pltpu.async_copy(ones, deg_sp.at[didx.at[g * NB + b]], sem, add=True)
        for b in range(NB):
            pltpu.make_async_copy(ones, deg_sp.at[didx.at[g * NB + b]], sem).wait()
        return 0

    lax.fori_loop(0, GROUPS, step, 0)

    plsc.subcore_barrier()

    @pl.when(sid < NS - 1)
    def _omain():
        pltpu.sync_copy(
            deg_sp.at[pl.ds(sid * R_MAIN, R_MAIN)],
            out_hbm.at[cid, pl.ds(sid * R_MAIN, R_MAIN)],
        )

    @pl.when(sid == NS - 1)
    def _olast():
        pltpu.sync_copy(
            deg_sp.at[pl.ds((NS - 1) * R_MAIN, R_LAST)],
            out_hbm.at[cid, pl.ds((NS - 1) * R_MAIN, R_LAST)],
        )


NB_AGG = 5
GROUPS_AGG = N_CHUNKS // NB_AGG


@functools.cache
def _make_agg(D):
    def agg(src_hbm, dst_hbm, tab_hbm, out_hbm, acc_sp, zbuf, sidx, didx,
            *rest):
        bufs = list(rest[:NB_AGG])
        gsem = list(rest[NB_AGG + 1:2 * NB_AGG + 1])
        ssem = list(rest[2 * NB_AGG + 1:])
        psem = rest[NB_AGG]
        cid = lax.axis_index("c")
        sid = lax.axis_index("s")
        wid = cid * NS + sid

        z16 = jnp.zeros((16,), jnp.float32)

        pi = pltpu.async_copy(src_hbm.at[wid], sidx, gsem[0])
        pj = pltpu.async_copy(dst_hbm.at[wid], didx, gsem[1])

        def fill_z(i, _):
            for j in range(D // 16):
                zbuf[i, pl.ds(j * 16, 16)] = z16
            return 0

        lax.fori_loop(0, R_MAIN, fill_z, 0)

        @pl.when(sid < NS - 1)
        def _zmain():
            pltpu.async_copy(zbuf, acc_sp.at[pl.ds(sid * R_MAIN, R_MAIN)], psem)

        @pl.when(sid == NS - 1)
        def _zlast():
            pltpu.async_copy(
                zbuf.at[pl.ds(0, R_LAST)],
                acc_sp.at[pl.ds((NS - 1) * R_MAIN, R_LAST)],
                psem,
            )

        pi.wait()
        pj.wait()

        @pl.when(sid < NS - 1)
        def _zmainw():
            pltpu.make_async_copy(
                zbuf, acc_sp.at[pl.ds(sid * R_MAIN, R_MAIN)], psem
            ).wait()

        @pl.when(sid == NS - 1)
        def _zlastw():
            pltpu.make_async_copy(
                zbuf.at[pl.ds(0, R_LAST)],
                acc_sp.at[pl.ds((NS - 1) * R_MAIN, R_LAST)],
                psem,
            ).wait()

        plsc.subcore_barrier()

        for b in range(NB_AGG):
            pltpu.async_copy(tab_hbm.at[sidx.at[b]], bufs[b], gsem[b])

        def step(g, _):
            base = g * NB_AGG
            for b in range(NB_AGG):
                k = base + b
                pltpu.make_async_copy(tab_hbm.at[sidx.at[k]], bufs[b], gsem[b]).wait()
                pltpu.async_copy(bufs[b], acc_sp.at[didx.at[k]], ssem[b], add=True)
            for b in range(NB_AGG):
                k = base + b
                pltpu.make_async_copy(bufs[b], acc_sp.at[didx.at[k]], ssem[b]).wait()

                def refill(b=b, k=k):
                    pltpu.async_copy(
                        tab_hbm.at[sidx.at[k + NB_AGG]], bufs[b], gsem[b]
                    )

                pl.when(g < GROUPS_AGG - 1)(refill)
            return 0

        lax.fori_loop(0, GROUPS_AGG, step, 0)

        plsc.subcore_barrier()

        @pl.when(sid < NS - 1)
        def _omain():
            pltpu.sync_copy(
                acc_sp.at[pl.ds(sid * R_MAIN, R_MAIN)],
                out_hbm.at[cid, pl.ds(sid * R_MAIN, R_MAIN)],
            )

        @pl.when(sid == NS - 1)
        def _olast():
            pltpu.sync_copy(
                acc_sp.at[pl.ds((NS - 1) * R_MAIN, R_LAST)],
                out_hbm.at[cid, pl.ds((NS - 1) * R_MAIN, R_LAST)],
            )

    return functools.partial(
        pl.kernel,
        out_type=jax.ShapeDtypeStruct((NC, NN, D), jnp.float32),
        mesh=_get_mesh(),
        scratch_types=(
            [
                pltpu.VMEM_SHARED((NN, D), jnp.float32),
                pltpu.VMEM((R_MAIN, D), jnp.float32),
                pltpu.VMEM((N_CHUNKS, CHUNK), jnp.int32),
                pltpu.VMEM((N_CHUNKS, CHUNK), jnp.int32),
            ]
            + [pltpu.VMEM((CHUNK, D), jnp.float32) for _ in range(NB_AGG)]
            + [pltpu.SemaphoreType.DMA for _ in range(2 * NB_AGG + 1)]
        ),
        compiler_params=pltpu.CompilerParams(use_tc_tiling_on_sc=False),
    )(agg)


PAIR_SPAN = 3200
PP_PAD = NW * PAIR_SPAN
PAIR_VREGS = PAIR_SPAN // 16


@functools.cache
def _get_pair_kernel():
    return functools.partial(
        pl.kernel,
        out_type=jax.ShapeDtypeStruct((NW, PAIR_SPAN), jnp.float32),
        mesh=_get_mesh(),
        scratch_types=[
            pltpu.VMEM((NN, 2), jnp.float32),
            pltpu.VMEM((PAIR_SPAN,), jnp.int32),
            pltpu.VMEM((PAIR_SPAN,), jnp.int32),
            pltpu.VMEM((PAIR_SPAN,), jnp.float32),
        ],
        compiler_params=pltpu.CompilerParams(
            use_tc_tiling_on_sc=False, needs_layout_passes=False
        ),
    )(_pair_body)


def _pair_body(ab_hbm, i0_hbm, i1_hbm, out_hbm, tab, b0, b1v, ob):
    cid = lax.axis_index("c")
    sid = lax.axis_index("s")
    wid = cid * NS + sid

    pltpu.sync_copy(ab_hbm, tab)
    pltpu.sync_copy(i0_hbm.at[wid], b0)
    pltpu.sync_copy(i1_hbm.at[wid], b1v)

    zi = jnp.zeros((16,), jnp.int32)
    oi = jnp.ones((16,), jnp.int32)

    def step(q, _):
        for j in range(8):
            off = (q * 8 + j) * 16
            ia = b0[pl.ds(off, 16)]
            ib = b1v[pl.ds(off, 16)]
            a = plsc.load_gather(tab, [ia, zi])
            b = plsc.load_gather(tab, [ib, oi])
            s = 1.0 / (1.0 + jnp.exp(-(a + b)))
            ob[pl.ds(off, 16)] = s
        return 0

    lax.fori_loop(0, PAIR_VREGS // 8, step, 0)
    pltpu.sync_copy(ob, out_hbm.at[wid])


BM = 400
GM = NN // BM


def _mm1_body(x_ref, w_ref, xw_ref):
    xw_ref[...] = jnp.dot(x_ref[...], w_ref[...], preferred_element_type=jnp.float32)


_mm1 = pl.pallas_call(
    _mm1_body,
    grid=(GM,),
    in_specs=[
        pl.BlockSpec((BM, NN), lambda i: (i, 0)),
        pl.BlockSpec((NN, 32), lambda i: (0, 0)),
    ],
    out_specs=pl.BlockSpec((BM, 32), lambda i: (i, 0)),
    out_shape=jax.ShapeDtypeStruct((NN, 32), jnp.float32),
)


def _scale_body(cnt_ref, xw_ref, xs_ref, dis_ref):
    cnt = cnt_ref[0, :, 0:1] + cnt_ref[1, :, 0:1] + 1.0
    dis = lax.rsqrt(cnt)
    xs_ref[...] = xw_ref[...] * dis
    dis_ref[...] = dis


_scale = pl.pallas_call(
    _scale_body,
    out_shape=[
        jax.ShapeDtypeStruct((NN, 32), jnp.float32),
        jax.ShapeDtypeStruct((NN, 1), jnp.float32),
    ],
)


def _mid_body(ap_ref, xs_ref, dis_ref, b1_ref, w2_ref, ys_ref):
    dis = dis_ref[...]
    h1 = jnp.maximum(
        dis * (ap_ref[0] + ap_ref[1] + xs_ref[...]) + b1_ref[...], 0.0
    )
    ys_ref[...] = jnp.dot(h1, w2_ref[...], preferred_element_type=jnp.float32) * dis


_mid = pl.pallas_call(
    _mid_body,
    out_shape=jax.ShapeDtypeStruct((NN, 16), jnp.float32),
)


def _head_body(ap_ref, ys_ref, dis_ref, b2_ref, wu_ref, bv_ref, ab_ref):
    dis = dis_ref[...]
    h2 = dis * (ap_ref[0] + ap_ref[1] + ys_ref[...]) + b2_ref[...]
    ab_ref[...] = (
        jnp.dot(h2, wu_ref[...], preferred_element_type=jnp.float32) + bv_ref[...]
    )


_head = pl.pallas_call(
    _head_body,
    out_shape=jax.ShapeDtypeStruct((NN, 2), jnp.float32),
)


def kernel(x, edge_index, pair_index, W1, b1, W2, b2, Wfc, bfc):
    src3 = edge_index[0].reshape(NW, N_CHUNKS, CHUNK)
    dst3 = edge_index[1].reshape(NW, N_CHUNKS, CHUNK)
    pad = (0, PP_PAD - PP)
    i0 = jnp.pad(pair_index[:, 0], pad).reshape(NW, PAIR_SPAN)
    i1 = jnp.pad(pair_index[:, 1], pad).reshape(NW, PAIR_SPAN)

    cnt16 = _get_deg_kernel()(dst3)
    xw = _mm1(x, W1)
    xs, dis = _scale(cnt16, xw)
    accp = _make_agg(32)(src3, dst3, xs)
    ys = _mid(accp, xs, dis, b1.reshape(1, 32), W2)
    acc2p = _make_agg(16)(src3, dst3, ys)

    wu = jnp.concatenate([Wfc[:16], Wfc[16:]], axis=1)
    bv = jnp.stack([bfc[0], jnp.zeros((), jnp.float32)]).reshape(1, 2)
    ab = _head(acc2p, ys, dis, b2.reshape(1, 16), wu, bv)

    out = _get_pair_kernel()(ab, i0, i1)
    return out.reshape(PP_PAD)[:PP].reshape(PP, 1)


# speedup vs baseline: 29.0561x; 1.0029x over previous
import functools

import jax
import jax.numpy as jnp
from jax import lax
from jax.experimental import pallas as pl
from jax.experimental.pallas import tpu as pltpu
from jax.experimental.pallas import tpu_sc as plsc

NN = 10000
EE = 320000
PP = 100000

NC = 2
NS = 16
NW = NC * NS

E_PER_W = EE // NW
CHUNK = 80
N_CHUNKS = E_PER_W // CHUNK
R_MAIN = 632
R_LAST = NN - (NS - 1) * R_MAIN

@functools.cache
def _get_mesh():
    return plsc.VectorSubcoreMesh(
        core_axis_name="c", subcore_axis_name="s", num_cores=NC, num_subcores=NS
    )


NB = 5
GROUPS = N_CHUNKS // NB


@functools.cache
def _get_deg_kernel():
    return functools.partial(
        pl.kernel,
        out_type=jax.ShapeDtypeStruct((NC, NN, 16), jnp.float32),
        mesh=_get_mesh(),
        scratch_types=[
            pltpu.VMEM_SHARED((NN, 16), jnp.float32),
            pltpu.VMEM((R_MAIN, 16), jnp.float32),
            pltpu.VMEM((CHUNK, 16), jnp.float32),
            pltpu.VMEM((N_CHUNKS, CHUNK), jnp.int32),
            pltpu.SemaphoreType.DMA,
        ],
        compiler_params=pltpu.CompilerParams(use_tc_tiling_on_sc=False),
    )(_deg_body)


def _deg_body(dst_hbm, out_hbm, deg_sp, zbuf, ones, didx, sem):
    cid = lax.axis_index("c")
    sid = lax.axis_index("s")
    wid = cid * NS + sid

    z16 = jnp.zeros((16,), jnp.float32)
    o16 = jnp.ones((16,), jnp.float32)

    def fill_z(i, _):
        zbuf[i, :] = z16
        return 0

    lax.fori_loop(0, R_MAIN, fill_z, 0)

    def fill_o(i, _):
        ones[i, :] = o16
        return 0

    lax.fori_loop(0, CHUNK, fill_o, 0)

    pltpu.sync_copy(dst_hbm.at[wid], didx)

    @pl.when(sid < NS - 1)
    def _zmain():
        pltpu.sync_copy(zbuf, deg_sp.at[pl.ds(sid * R_MAIN, R_MAIN)])

    @pl.when(sid == NS - 1)
    def _zlast():
        pltpu.sync_copy(
            zbuf.at[pl.ds(0, R_LAST)],
            deg_sp.at[pl.ds((NS - 1) * R_MAIN, R_LAST)],
        )

    plsc.subcore_barrier()

    def step(g, _):
        for b in range(NB):
            pltpu.async_copy(ones, deg_sp.at[didx.at[g * NB + b]], sem, add=True)
        for b in range(NB):
            pltpu.make_async_copy(ones, deg_sp.at[didx.at[g * NB + b]], sem).wait()
        return 0

    lax.fori_loop(0, GROUPS, step, 0)

    plsc.subcore_barrier()

    @pl.when(sid < NS - 1)
    def _omain():
        pltpu.sync_copy(
            deg_sp.at[pl.ds(sid * R_MAIN, R_MAIN)],
            out_hbm.at[cid, pl.ds(sid * R_MAIN, R_MAIN)],
        )

    @pl.when(sid == NS - 1)
    def _olast():
        pltpu.sync_copy(
            deg_sp.at[pl.ds((NS - 1) * R_MAIN, R_LAST)],
            out_hbm.at[cid, pl.ds((NS - 1) * R_MAIN, R_LAST)],
        )


NB_AGG = 5
GROUPS_AGG = N_CHUNKS // NB_AGG


@functools.cache
def _make_agg(D):
    def agg(src_hbm, dst_hbm, tab_hbm, out_hbm, acc_sp, zbuf, sidx, didx,
            *rest):
        bufs = list(rest[:NB_AGG])
        gsem = list(rest[NB_AGG + 1:2 * NB_AGG + 1])
        ssem = list(rest[2 * NB_AGG + 1:])
        psem = rest[NB_AGG]
        cid = lax.axis_index("c")
        sid = lax.axis_index("s")
        wid = cid * NS + sid

        z16 = jnp.zeros((16,), jnp.float32)

        pi = pltpu.async_copy(src_hbm.at[wid], sidx, gsem[0])
        pj = pltpu.async_copy(dst_hbm.at[wid], didx, gsem[1])

        def fill_z(i, _):
            for j in range(D // 16):
                zbuf[i, pl.ds(j * 16, 16)] = z16
            return 0

        lax.fori_loop(0, R_MAIN, fill_z, 0)

        @pl.when(sid < NS - 1)
        def _zmain():
            pltpu.async_copy(zbuf, acc_sp.at[pl.ds(sid * R_MAIN, R_MAIN)], psem)

        @pl.when(sid == NS - 1)
        def _zlast():
            pltpu.async_copy(
                zbuf.at[pl.ds(0, R_LAST)],
                acc_sp.at[pl.ds((NS - 1) * R_MAIN, R_LAST)],
                psem,
            )

        pi.wait()
        pj.wait()

        @pl.when(sid < NS - 1)
        def _zmainw():
            pltpu.make_async_copy(
                zbuf, acc_sp.at[pl.ds(sid * R_MAIN, R_MAIN)], psem
            ).wait()

        @pl.when(sid == NS - 1)
        def _zlastw():
            pltpu.make_async_copy(
                zbuf.at[pl.ds(0, R_LAST)],
                acc_sp.at[pl.ds((NS - 1) * R_MAIN, R_LAST)],
                psem,
            ).wait()

        plsc.subcore_barrier()

        for b in range(NB_AGG):
            pltpu.async_copy(tab_hbm.at[sidx.at[b]], bufs[b], gsem[b])

        def step(g, _):
            base = g * NB_AGG
            for b in range(NB_AGG):
                k = base + b
                pltpu.make_async_copy(tab_hbm.at[sidx.at[k]], bufs[b], gsem[b]).wait()
                pltpu.async_copy(bufs[b], acc_sp.at[didx.at[k]], ssem[b], add=True)
            for b in range(NB_AGG):
                k = base + b
                pltpu.make_async_copy(bufs[b], acc_sp.at[didx.at[k]], ssem[b]).wait()

                def refill(b=b, k=k):
                    pltpu.async_copy(
                        tab_hbm.at[sidx.at[k + NB_AGG]], bufs[b], gsem[b]
                    )

                pl.when(g < GROUPS_AGG - 1)(refill)
            return 0

        lax.fori_loop(0, GROUPS_AGG, step, 0)

        plsc.subcore_barrier()

        @pl.when(sid < NS - 1)
        def _omain():
            pltpu.sync_copy(
                acc_sp.at[pl.ds(sid * R_MAIN, R_MAIN)],
                out_hbm.at[cid, pl.ds(sid * R_MAIN, R_MAIN)],
            )

        @pl.when(sid == NS - 1)
        def _olast():
            pltpu.sync_copy(
                acc_sp.at[pl.ds((NS - 1) * R_MAIN, R_LAST)],
                out_hbm.at[cid, pl.ds((NS - 1) * R_MAIN, R_LAST)],
            )

    return functools.partial(
        pl.kernel,
        out_type=jax.ShapeDtypeStruct((NC, NN, D), jnp.float32),
        mesh=_get_mesh(),
        scratch_types=(
            [
                pltpu.VMEM_SHARED((NN, D), jnp.float32),
                pltpu.VMEM((R_MAIN, D), jnp.float32),
                pltpu.VMEM((N_CHUNKS, CHUNK), jnp.int32),
                pltpu.VMEM((N_CHUNKS, CHUNK), jnp.int32),
            ]
            + [pltpu.VMEM((CHUNK, D), jnp.float32) for _ in range(NB_AGG)]
            + [pltpu.SemaphoreType.DMA for _ in range(2 * NB_AGG + 1)]
        ),
        compiler_params=pltpu.CompilerParams(use_tc_tiling_on_sc=False),
    )(agg)


PAIR_SPAN = 3200
PP_PAD = NW * PAIR_SPAN
PAIR_VREGS = PAIR_SPAN // 16


@functools.cache
def _get_pair_kernel():
    return functools.partial(
        pl.kernel,
        out_type=jax.ShapeDtypeStruct((NW, PAIR_SPAN), jnp.float32),
        mesh=_get_mesh(),
        scratch_types=[
            pltpu.VMEM((NN, 2), jnp.float32),
            pltpu.VMEM((PAIR_SPAN,), jnp.int32),
            pltpu.VMEM((PAIR_SPAN,), jnp.int32),
            pltpu.VMEM((PAIR_SPAN,), jnp.float32),
        ],
        compiler_params=pltpu.CompilerParams(
            use_tc_tiling_on_sc=False, needs_layout_passes=False
        ),
    )(_pair_body)


def _pair_body(ab_hbm, i0_hbm, i1_hbm, out_hbm, tab, b0, b1v, ob):
    cid = lax.axis_index("c")
    sid = lax.axis_index("s")
    wid = cid * NS + sid

    pltpu.sync_copy(ab_hbm, tab)
    pltpu.sync_copy(i0_hbm.at[wid], b0)
    pltpu.sync_copy(i1_hbm.at[wid], b1v)

    zi = jnp.zeros((16,), jnp.int32)
    oi = jnp.ones((16,), jnp.int32)

    def step(q, _):
        for j in range(8):
            off = (q * 8 + j) * 16
            ia = b0[pl.ds(off, 16)]
            ib = b1v[pl.ds(off, 16)]
            a = plsc.load_gather(tab, [ia, zi])
            b = plsc.load_gather(tab, [ib, oi])
            s = 1.0 / (1.0 + jnp.exp(-(a + b)))
            ob[pl.ds(off, 16)] = s
        return 0

    lax.fori_loop(0, PAIR_VREGS // 8, step, 0)
    pltpu.sync_copy(ob, out_hbm.at[wid])


BM = 400
GM = NN // BM


def _mm1_body(x_ref, w_ref, xw_ref):
    xw_ref[...] = jnp.dot(x_ref[...], w_ref[...], preferred_element_type=jnp.float32)


_mm1 = pl.pallas_call(
    _mm1_body,
    grid=(GM,),
    in_specs=[
        pl.BlockSpec((BM, NN), lambda i: (i, 0)),
        pl.BlockSpec((NN, 32), lambda i: (0, 0)),
    ],
    out_specs=pl.BlockSpec((BM, 32), lambda i: (i, 0)),
    out_shape=jax.ShapeDtypeStruct((NN, 32), jnp.float32),
)


def _scale_body(cnt_ref, xw_ref, xs_ref, dis_ref):
    cnt = cnt_ref[0, :, 0:1] + cnt_ref[1, :, 0:1] + 1.0
    dis = lax.rsqrt(cnt)
    xs_ref[...] = xw_ref[...] * dis
    dis_ref[...] = dis


_scale = pl.pallas_call(
    _scale_body,
    out_shape=[
        jax.ShapeDtypeStruct((NN, 32), jnp.float32),
        jax.ShapeDtypeStruct((NN, 1), jnp.float32),
    ],
)


def _mid_body(ap_ref, xs_ref, dis_ref, b1_ref, w2_ref, ys_ref):
    dis = dis_ref[...]
    h1 = jnp.maximum(
        dis * (ap_ref[0] + ap_ref[1] + xs_ref[...]) + b1_ref[...], 0.0
    )
    ys_ref[...] = jnp.dot(h1, w2_ref[...], preferred_element_type=jnp.float32) * dis


_mid = pl.pallas_call(
    _mid_body,
    out_shape=jax.ShapeDtypeStruct((NN, 16), jnp.float32),
)


def _head_body(ap_ref, ys_ref, dis_ref, b2_ref, wu_ref, bv_ref, ab_ref):
    dis = dis_ref[...]
    h2 = dis * (ap_ref[0] + ap_ref[1] + ys_ref[...]) + b2_ref[...]
    ab_ref[...] = (
        jnp.dot(h2, wu_ref[...], preferred_element_type=jnp.float32) + bv_ref[...]
    )


_head = pl.pallas_call(
    _head_body,
    out_shape=jax.ShapeDtypeStruct((NN, 2), jnp.float32),
)


def kernel(x, edge_index, pair_index, W1, b1, W2, b2, Wfc, bfc):
    src3 = edge_index[0].reshape(NW, N_CHUNKS, CHUNK)
    dst3 = edge_index[1].reshape(NW, N_CHUNKS, CHUNK)
    pad = (0, PP_PAD - PP)
    i0 = jnp.pad(pair_index[:, 0], pad).reshape(NW, PAIR_SPAN)
    i1 = jnp.pad(pair_index[:, 1], pad).reshape(NW, PAIR_SPAN)

    cnt16 = _get_deg_kernel()(dst3)
    xw = _mm1(x, W1)
    xs, dis = _scale(cnt16, xw)
    accp = _make_agg(32)(src3, dst3, xs)
    ys = _mid(accp, xs, dis, b1.reshape(1, 32), W2)
    acc2p = _make_agg(16)(src3, dst3, ys)

    wu = jnp.concatenate([Wfc[:16], Wfc[16:]], axis=1)
    bv = jnp.stack([bfc[0], jnp.zeros((), jnp.float32)]).reshape(1, 2)
    ab = _head(acc2p, ys, dis, b2.reshape(1, 16), wu, bv)

    out = _get_pair_kernel()(ab, i0, i1)
    return out.reshape(PP_PAD)[:PP].reshape(PP, 1)
